# baseline (speedup 1.0000x reference)
import numpy as np
import jax
import jax.numpy as jnp
from jax.sharding import Mesh, PartitionSpec as P
from jax.experimental import pallas as pl
from jax.experimental.pallas import tpu as pltpu


def _geo_conv_kernel(x_ref, adj_ref, w_ref, o_ref):
    y = jax.lax.dot_general(
        adj_ref[...], x_ref[...],
        dimension_numbers=(((1,), (0,)), ((), ())),
        preferred_element_type=jnp.float32,
    )
    o_ref[...] = 0.75 * y + jax.lax.dot_general(
        y, w_ref[...],
        dimension_numbers=(((1,), (1,)), ((), ())),
        preferred_element_type=jnp.float32,
    )


def _geo_conv_shard(x, adj, weight):
    m, n = adj.shape
    d = x.shape[1]
    bm = 200 if m % 200 == 0 else m
    return pl.pallas_call(
        _geo_conv_kernel,
        grid=(m // bm,),
        in_specs=[
            pl.BlockSpec((n, d), lambda i: (0, 0)),
            pl.BlockSpec((bm, n), lambda i: (i, 0)),
            pl.BlockSpec((d, d), lambda i: (0, 0)),
        ],
        out_specs=pl.BlockSpec((bm, d), lambda i: (i, 0)),
        out_shape=jax.ShapeDtypeStruct((m, d), jnp.float32),
        compiler_params=pltpu.CompilerParams(
            dimension_semantics=("arbitrary",),
        ),
    )(x, adj, weight)


def kernel(x, adj, weight):
    devs = jax.devices()
    ndev = 2 if (len(devs) >= 2 and adj.shape[0] % 2 == 0) else 1
    if ndev == 1:
        return _geo_conv_shard(x, adj, weight)
    mesh = Mesh(np.array(devs[:ndev]), ("i",))
    f = jax.shard_map(
        _geo_conv_shard,
        mesh=mesh,
        in_specs=(P(), P("i", None), P()),
        out_specs=P("i", None),
        check_vma=False,
    )
    return f(x, adj, weight)


# speedup vs baseline: 5.7317x; 5.7317x over previous
import jax
import jax.numpy as jnp
from jax.experimental import pallas as pl
from jax.experimental.pallas import tpu as pltpu


def _geo_conv_kernel(x_ref, adj_ref, w_ref, o_ref):
    y = jax.lax.dot_general(
        adj_ref[...], x_ref[...],
        dimension_numbers=(((1,), (0,)), ((), ())),
        preferred_element_type=jnp.float32,
    )
    o_ref[...] = 0.75 * y + jax.lax.dot_general(
        y, w_ref[...],
        dimension_numbers=(((1,), (1,)), ((), ())),
        preferred_element_type=jnp.float32,
    )


def kernel(x, adj, weight):
    n, d = x.shape
    bm = 400 if n % 400 == 0 else n
    grid = (n // bm,)
    return pl.pallas_call(
        _geo_conv_kernel,
        grid=grid,
        in_specs=[
            pl.BlockSpec((n, d), lambda i: (0, 0)),
            pl.BlockSpec((bm, n), lambda i: (i, 0)),
            pl.BlockSpec((d, d), lambda i: (0, 0)),
        ],
        out_specs=pl.BlockSpec((bm, d), lambda i: (i, 0)),
        out_shape=jax.ShapeDtypeStruct((n, d), jnp.float32),
        compiler_params=pltpu.CompilerParams(
            dimension_semantics=("parallel",),
        ),
    )(x, adj, weight)
